# baseline (device time: 887137 ns/iter reference)
import functools

import jax
import jax.numpy as jnp
from jax import lax
from jax.experimental import pallas as pl
from jax.experimental.pallas import tpu as pltpu

N_DEV = 32
SQ = 512
D = 1024
HQ = 8
HKV = 2
DH = 128
SKV_LOC = 2048
SCALE = 0.08838834764831843


def _body(x_ref, wq_ref, wo_ref, k_ref, v_ref, out_ref,
          comm_o, comm_l, acc_o, acc_l,
          send_o_sems, recv_o_sems, send_l_sems, recv_l_sems,
          credit_sem):
    my = lax.axis_index("i")
    left = (my - 1) % N_DEV
    right = (my + 1) % N_DEV

    barrier_sem = pltpu.get_barrier_semaphore()
    for nbr in (left, right):
        pl.semaphore_signal(barrier_sem, inc=1, device_id=(nbr,),
                            device_id_type=pl.DeviceIdType.MESH)
    pl.semaphore_wait(barrier_sem, 2)

    q = jnp.dot(x_ref[...], wq_ref[...], preferred_element_type=jnp.float32)

    for h in range(HQ):
        g = h // 4
        qh = q[:, h * DH:(h + 1) * DH]
        kg = k_ref[:, g, :]
        s = lax.dot_general(qh, kg, (((1,), (1,)), ((), ())),
                            preferred_element_type=jnp.float32)
        p = jnp.exp(s * SCALE)
        comm_l[0, :, h:h + 1] = jnp.sum(p, axis=1, keepdims=True)
        comm_o[0, h] = lax.dot_general(p, v_ref[:, g, :],
                                       (((1,), (0,)), ((), ())),
                                       preferred_element_type=jnp.float32)

    acc_o[...] = comm_o[0]
    acc_l[...] = comm_l[0]

    for hop in range(N_DEV - 1):
        ss = hop % 2
        rs = (hop + 1) % 2
        if hop >= 2:
            pl.semaphore_wait(credit_sem, 1)
        rdma_o = pltpu.make_async_remote_copy(
            src_ref=comm_o.at[ss], dst_ref=comm_o.at[rs],
            send_sem=send_o_sems.at[ss], recv_sem=recv_o_sems.at[rs],
            device_id=(right,), device_id_type=pl.DeviceIdType.MESH)
        rdma_l = pltpu.make_async_remote_copy(
            src_ref=comm_l.at[ss], dst_ref=comm_l.at[rs],
            send_sem=send_l_sems.at[ss], recv_sem=recv_l_sems.at[rs],
            device_id=(right,), device_id_type=pl.DeviceIdType.MESH)
        rdma_o.start()
        rdma_l.start()
        rdma_o.wait()
        rdma_l.wait()
        acc_o[...] += comm_o[rs]
        acc_l[...] += comm_l[rs]
        if hop < N_DEV - 3:
            pl.semaphore_signal(credit_sem, inc=1, device_id=(left,),
                                device_id_type=pl.DeviceIdType.MESH)

    for h in range(HQ):
        o_h = acc_o[h] / acc_l[:, h:h + 1]
        part = jnp.dot(o_h, wo_ref[h * DH:(h + 1) * DH, :],
                       preferred_element_type=jnp.float32)
        if h == 0:
            out_ref[...] = part
        else:
            out_ref[...] += part


def kernel(x, Wq, Wo, K_ext, V_ext):
    x2 = x.reshape(SQ, D)
    k2 = K_ext.reshape(SKV_LOC, HKV, DH)
    v2 = V_ext.reshape(SKV_LOC, HKV, DH)

    out2 = pl.pallas_call(
        _body,
        out_shape=jax.ShapeDtypeStruct((SQ, D), jnp.float32),
        in_specs=[pl.BlockSpec(memory_space=pltpu.VMEM)] * 5,
        out_specs=pl.BlockSpec(memory_space=pltpu.VMEM),
        scratch_shapes=[
            pltpu.VMEM((2, HQ, SQ, DH), jnp.float32),
            pltpu.VMEM((2, SQ, HQ), jnp.float32),
            pltpu.VMEM((HQ, SQ, DH), jnp.float32),
            pltpu.VMEM((SQ, HQ), jnp.float32),
            pltpu.SemaphoreType.DMA((2,)),
            pltpu.SemaphoreType.DMA((2,)),
            pltpu.SemaphoreType.DMA((2,)),
            pltpu.SemaphoreType.DMA((2,)),
            pltpu.SemaphoreType.REGULAR,
        ],
        compiler_params=pltpu.CompilerParams(collective_id=0),
    )(x2, Wq, Wo, k2, v2)

    return out2.reshape(1, SQ, D)


# device time: 129837 ns/iter; 6.8327x vs baseline; 6.8327x over previous
import jax
import jax.numpy as jnp
from jax import lax
from jax.experimental import pallas as pl
from jax.experimental.pallas import tpu as pltpu

N_DEV = 32
SQ = 512
D = 1024
HQ = 8
HKV = 2
DH = 128
SKV_LOC = 2048
SCALE = 0.08838834764831843
ROUNDS = 5
RS_OFF = (0, 256, 384, 448, 480)

_MESH = pl.DeviceIdType.MESH


def _body(x_ref, wq_ref, wo_ref, k_ref, v_ref, out_ref,
          acc_o, acc_l, stage_o, stage_l,
          rs_o_send, rs_o_recv, rs_l_send, rs_l_recv, ag_send, ag_recv):
    my = lax.axis_index("i")

    barrier_sem = pltpu.get_barrier_semaphore()
    for r in range(ROUNDS):
        pl.semaphore_signal(barrier_sem, inc=1, device_id=(my ^ (1 << r),),
                            device_id_type=_MESH)
    pl.semaphore_wait(barrier_sem, ROUNDS)

    q = jnp.dot(x_ref[...], wq_ref[...], preferred_element_type=jnp.float32)

    for h in range(HQ):
        g = h // 4
        qh = q[:, h * DH:(h + 1) * DH]
        kg = k_ref[:, g, :]
        s = lax.dot_general(qh, kg, (((1,), (1,)), ((), ())),
                            preferred_element_type=jnp.float32)
        p = jnp.exp(s * SCALE)
        acc_l[:, h:h + 1] = jnp.sum(p, axis=1, keepdims=True)
        acc_o[:, h, :] = lax.dot_general(p, v_ref[:, g, :],
                                         (((1,), (0,)), ((), ())),
                                         preferred_element_type=jnp.float32)

    lo = my * 0
    for r in range(ROUNDS):
        d = 16 >> r
        half = 256 >> r
        p_dev = my ^ d
        bit = (my // d) % 2
        keep_lo = lo + bit * half
        send_lo = lo + (1 - bit) * half
        off = RS_OFF[r]
        ro = pltpu.make_async_remote_copy(
            src_ref=acc_o.at[pl.ds(send_lo, half)],
            dst_ref=stage_o.at[pl.ds(off, half)],
            send_sem=rs_o_send.at[r], recv_sem=rs_o_recv.at[r],
            device_id=(p_dev,), device_id_type=_MESH)
        rl = pltpu.make_async_remote_copy(
            src_ref=acc_l.at[pl.ds(send_lo, half)],
            dst_ref=stage_l.at[pl.ds(off, half)],
            send_sem=rs_l_send.at[r], recv_sem=rs_l_recv.at[r],
            device_id=(p_dev,), device_id_type=_MESH)
        ro.start()
        rl.start()
        ro.wait()
        rl.wait()
        acc_o[pl.ds(keep_lo, half)] += stage_o[pl.ds(off, half)]
        acc_l[pl.ds(keep_lo, half)] += stage_l[pl.ds(off, half)]
        lo = keep_lo

    part = None
    for h in range(HQ):
        o_h = acc_o[pl.ds(lo, 16), h, :]
        l_h = acc_l[pl.ds(lo, 16), h:h + 1]
        t = jnp.dot(o_h / l_h, wo_ref[h * DH:(h + 1) * DH, :],
                    preferred_element_type=jnp.float32)
        part = t if part is None else part + t
    out_ref[pl.ds(lo, 16), :] = part

    for r in range(ROUNDS):
        d = 1 << r
        blk = 16 << r
        p_dev = my ^ d
        cur_lo = (my - my % d) * 16
        ag = pltpu.make_async_remote_copy(
            src_ref=out_ref.at[pl.ds(cur_lo, blk)],
            dst_ref=out_ref.at[pl.ds(cur_lo, blk)],
            send_sem=ag_send.at[r], recv_sem=ag_recv.at[r],
            device_id=(p_dev,), device_id_type=_MESH)
        ag.start()
        ag.wait()


def kernel(x, Wq, Wo, K_ext, V_ext):
    x2 = x.reshape(SQ, D)
    k2 = K_ext.reshape(SKV_LOC, HKV, DH)
    v2 = V_ext.reshape(SKV_LOC, HKV, DH)

    out2 = pl.pallas_call(
        _body,
        out_shape=jax.ShapeDtypeStruct((SQ, D), jnp.float32),
        in_specs=[pl.BlockSpec(memory_space=pltpu.VMEM)] * 5,
        out_specs=pl.BlockSpec(memory_space=pltpu.VMEM),
        scratch_shapes=[
            pltpu.VMEM((SQ, HQ, DH), jnp.float32),
            pltpu.VMEM((SQ, HQ), jnp.float32),
            pltpu.VMEM((496, HQ, DH), jnp.float32),
            pltpu.VMEM((496, HQ), jnp.float32),
            pltpu.SemaphoreType.DMA((ROUNDS,)),
            pltpu.SemaphoreType.DMA((ROUNDS,)),
            pltpu.SemaphoreType.DMA((ROUNDS,)),
            pltpu.SemaphoreType.DMA((ROUNDS,)),
            pltpu.SemaphoreType.DMA((ROUNDS,)),
            pltpu.SemaphoreType.DMA((ROUNDS,)),
        ],
        compiler_params=pltpu.CompilerParams(collective_id=0),
    )(x2, Wq, Wo, k2, v2)

    return out2.reshape(1, SQ, D)
